# baseline (speedup 1.0000x reference)
import jax
import jax.numpy as jnp
from jax.experimental import pallas as pl
from jax.experimental.pallas import tpu as pltpu

_K, _N, _D = 8, 2048, 128
_BI = 256


def _fused_kernel(theta_ref, T_ref, a_ref, x_ref, alpha_ref, W_ref, b_ref,
                  o_ref, s_ref):
    k = pl.program_id(1)

    @pl.when(k == 0)
    def _init():
        s_ref[...] = theta_ref[0] * T_ref[0]

    @pl.when(k > 0)
    def _accum():
        s_ref[...] += theta_ref[k] * T_ref[0]

    @pl.when(k == _K - 1)
    def _epilogue():
        q = s_ref[...] * a_ref[...]
        h = jnp.dot(q, x_ref[...], preferred_element_type=jnp.float32)
        h = jnp.where(h >= 0, h, alpha_ref[...] * h)
        o_ref[...] = jax.lax.dot_general(
            h, W_ref[...], (((1,), (1,)), ((), ())),
            preferred_element_type=jnp.float32) + b_ref[...]


def kernel(theta, T_slices, x, a, prelu_alpha, W, b):
    grid = (_N // _BI, _K)
    out = pl.pallas_call(
        _fused_kernel,
        grid=grid,
        in_specs=[
            pl.BlockSpec(memory_space=pltpu.SMEM),
            pl.BlockSpec((1, _BI, _N), lambda i, k: (k, i, 0)),
            pl.BlockSpec((_BI, _N), lambda i, k: (i, 0)),
            pl.BlockSpec((_N, _D), lambda i, k: (0, 0)),
            pl.BlockSpec((1, _D), lambda i, k: (0, 0)),
            pl.BlockSpec((_D, _D), lambda i, k: (0, 0)),
            pl.BlockSpec((1, _D), lambda i, k: (0, 0)),
        ],
        out_specs=pl.BlockSpec((_BI, _D), lambda i, k: (i, 0)),
        out_shape=jax.ShapeDtypeStruct((_N, _D), jnp.float32),
        scratch_shapes=[pltpu.VMEM((_BI, _N), jnp.float32)],
        compiler_params=pltpu.CompilerParams(
            dimension_semantics=("parallel", "arbitrary"),
        ),
    )(theta, T_slices, a, x, prelu_alpha.reshape(1, _D), W,
      b.reshape(1, _D))
    return out
